# baseline (speedup 1.0000x reference)
import jax
import jax.numpy as jnp
from jax.experimental import pallas as pl
from jax.experimental.pallas import tpu as pltpu

_IN, _H, _L, _S = 512, 512, 2, 3
_B, _T = 64, 512
_PENALTY = 0.7
_TC = 32
_G3 = 3 * _H


def _body(xb_ref, wlw_ref, wsel_ref, bsel_ref, blw_ref,
          wih0c_ref, whh0c_ref, bih0_ref, bhh0_ref,
          wih1_ref, whh1_ref, bih1_ref, bhh1_ref,
          out_ref, h_ref, lp_ref, gi_ref, gh_ref):
    t_base = pl.program_id(0) * _TC

    @pl.when(t_base == 0)
    def _init():
        h_ref[...] = jnp.zeros((2 * _B, _H), jnp.float32)
        lane = jax.lax.broadcasted_iota(jnp.int32, (1, 128), 1)
        lp_ref[...] = jnp.where(lane < _S, 0.0, -jnp.inf).astype(jnp.float32)

    def step(tl, carry):
        t = t_base + tl
        x_t = xb_ref[pl.ds(tl, 1)].reshape(_B, _IN)
        h2d = h_ref[...]
        h_bf = h2d.astype(jnp.bfloat16)

        energy = jnp.dot(h_bf, wlw_ref[...], preferred_element_type=jnp.float32)
        le = jnp.sum(energy + blw_ref[...], axis=-1, keepdims=True)
        le_f = le.astype(jnp.bfloat16).astype(jnp.float32)
        prod = le_f * h_bf.astype(jnp.float32)
        sh = (prod[:_B] + prod[_B:]).astype(jnp.bfloat16)
        logits = (jnp.dot(sh, wsel_ref[:_H], preferred_element_type=jnp.float32)
                  + jnp.dot(x_t, wsel_ref[_H:], preferred_element_type=jnp.float32)
                  + bsel_ref[...])
        slog = jnp.sum(logits, axis=0, keepdims=True)

        gi_ref[...] = jnp.dot(x_t, wih0c_ref[...],
                              preferred_element_type=jnp.float32)
        gh_ref[...] = jnp.dot(h_bf[:_B], whh0c_ref[...],
                              preferred_element_type=jnp.float32)

        s = slog + lp_ref[...]
        s0, s1, s2 = s[0, 0], s[0, 1], s[0, 2]
        cur = jnp.where((s0 >= s1) & (s0 >= s2), 0,
                        jnp.where(s1 >= s2, 1, 2)).astype(jnp.int32)
        cur = jnp.where(t == 0, 0, cur)

        lane = jax.lax.broadcasted_iota(jnp.int32, (1, 128), 1)
        lpn = lp_ref[...] + jnp.where(lane == cur, _LOG_PENALTY, 0.0)
        lp_ref[...] = lpn - jnp.max(lpn)

        h0 = h2d[:_B]
        h1 = h2d[_B:]
        off = cur * _G3

        def gates(gi, gh, h_prev):
            r = jax.nn.sigmoid(gi[:, :_H] + gh[:, :_H])
            z = jax.nn.sigmoid(gi[:, _H:2 * _H] + gh[:, _H:2 * _H])
            n = jnp.tanh(gi[:, 2 * _H:] + r * gh[:, 2 * _H:])
            return (1.0 - z) * n + z * h_prev

        gi0 = gi_ref[:, pl.ds(off, _G3)] + bih0_ref[pl.ds(cur, 1)].reshape(1, _G3)
        gh0 = gh_ref[:, pl.ds(off, _G3)] + bhh0_ref[pl.ds(cur, 1)].reshape(1, _G3)
        h0n = gates(gi0, gh0, h0)

        w = lambda ref: ref[pl.ds(cur, 1)].reshape(_IN, _G3)
        b = lambda ref: ref[pl.ds(cur, 1)].reshape(1, _G3)
        gi1 = jnp.dot(h0n.astype(jnp.bfloat16), w(wih1_ref),
                      preferred_element_type=jnp.float32) + b(bih1_ref)
        gh1 = jnp.dot(h_bf[_B:], w(whh1_ref),
                      preferred_element_type=jnp.float32) + b(bhh1_ref)
        h1n = gates(gi1, gh1, h1)

        h_ref[:_B] = h0n
        h_ref[_B:] = h1n
        out_ref[pl.ds(tl, 1)] = h1n.reshape(1, _B, _H)
        return carry

    jax.lax.fori_loop(0, _TC, step, 0)


_LOG_PENALTY = float(jnp.log(jnp.float32(_PENALTY)))


def kernel(x, Wih_first, Wih_rest, Whh, bih, bhh, Wlw, blw, Wsel, bsel):
    f32, bf16 = jnp.float32, jnp.bfloat16
    xb = jnp.swapaxes(x, 0, 1).astype(bf16)
    wlw_t = Wlw.T.astype(bf16)
    wsel_t = jnp.zeros((_H + _IN, 128), f32).at[:, :_S].set(Wsel.T).astype(bf16)
    bsel_p = jnp.full((1, 128), -jnp.inf, f32).at[0, :_S].set(bsel)
    blw_r = blw.reshape(1, _H)
    wih0c = Wih_first.transpose(2, 0, 1).reshape(_IN, _S * _G3).astype(bf16)
    whh0c = Whh[:, 0].transpose(2, 0, 1).reshape(_H, _S * _G3).astype(bf16)
    wih1 = Wih_rest[:, 0].transpose(0, 2, 1).astype(bf16)
    whh1 = Whh[:, 1].transpose(0, 2, 1).astype(bf16)
    bih0, bih1 = bih[:, 0][:, None, :], bih[:, 1][:, None, :]
    bhh0, bhh1 = bhh[:, 0][:, None, :], bhh[:, 1][:, None, :]

    full = lambda a: pl.BlockSpec(a.shape, lambda i: (0,) * a.ndim)
    outputs = pl.pallas_call(
        _body,
        grid=(_T // _TC,),
        in_specs=[pl.BlockSpec((_TC, _B, _IN), lambda i: (i, 0, 0))]
        + [full(a) for a in (wlw_t, wsel_t, bsel_p, blw_r,
                             wih0c, whh0c, bih0, bhh0, wih1, whh1, bih1, bhh1)],
        out_specs=pl.BlockSpec((_TC, _B, _H), lambda i: (i, 0, 0)),
        out_shape=jax.ShapeDtypeStruct((_T, _B, _H), f32),
        scratch_shapes=[pltpu.VMEM((2 * _B, _H), f32),
                        pltpu.VMEM((1, 128), f32),
                        pltpu.VMEM((_B, _S * _G3), f32),
                        pltpu.VMEM((_B, _S * _G3), f32)],
        compiler_params=pltpu.CompilerParams(
            dimension_semantics=("arbitrary",),
            vmem_limit_bytes=64 * 1024 * 1024,
        ),
    )(xb, wlw_t, wsel_t, bsel_p, blw_r,
      wih0c, whh0c, bih0, bhh0, wih1, whh1, bih1, bhh1)
    return outputs, outputs[-1]


# speedup vs baseline: 1.5360x; 1.5360x over previous
import jax
import jax.numpy as jnp
import numpy as np
from jax.experimental import pallas as pl
from jax.experimental.pallas import tpu as pltpu

_IN, _H, _L, _S = 512, 512, 2, 3
_B, _T = 64, 512
_PENALTY = 0.7
_LOG_PENALTY = float(np.log(np.float32(_PENALTY)))
_TC = 32
_G3 = 3 * _H


def _body(xb_ref, wlw_ref, w3sh_ref, w3x_ref, bsel64_ref, blw_ref,
          wih0_ref, whh0_ref, bih0_ref, bhh0_ref,
          wih1_ref, whh1_ref, bih1_ref, bhh1_ref,
          out_ref, h_ref, lp_ref):
    t_base = pl.program_id(0) * _TC
    f32, bf16 = jnp.float32, jnp.bfloat16

    @pl.when(t_base == 0)
    def _init():
        h_ref[...] = jnp.zeros((2 * _B, _H), f32)
        lp_ref[0] = 0.0
        lp_ref[1] = 0.0
        lp_ref[2] = 0.0

    def step(tl, hc):
        h0, h1 = hc
        t = t_base + tl
        x_t = xb_ref[pl.ds(tl, 1)].reshape(_B, _IN)

        xsum = jnp.sum(x_t.astype(f32), axis=0, keepdims=True)
        sx = jnp.sum(xsum * w3x_ref[...].astype(f32), axis=-1,
                     keepdims=True)

        h0b = h0.astype(bf16)
        h1b = h1.astype(bf16)
        hb2 = jnp.concatenate([h0b, h1b], axis=0)

        energy = jnp.dot(hb2, wlw_ref[...], preferred_element_type=f32)
        le = jnp.sum(energy + blw_ref[...], axis=-1, keepdims=True)
        le_f = le.astype(bf16).astype(f32)
        prod = le_f * hb2.astype(f32)
        sh = (prod[:_B] + prod[_B:]).astype(bf16)
        shsum = jnp.sum(sh.astype(f32), axis=0, keepdims=True)
        ssh = jnp.sum(shsum * w3sh_ref[...].astype(f32), axis=-1,
                      keepdims=True)
        srow = ssh + sx + bsel64_ref[...][:, :1]

        s0 = srow[0, 0] + lp_ref[0]
        s1 = srow[1, 0] + lp_ref[1]
        s2 = srow[2, 0] + lp_ref[2]
        cur = jnp.where((s0 >= s1) & (s0 >= s2), 0,
                        jnp.where(s1 >= s2, 1, 2)).astype(jnp.int32)
        cur = jnp.where(t == 0, 0, cur)

        l0 = lp_ref[0] + jnp.where(cur == 0, _LOG_PENALTY, 0.0)
        l1 = lp_ref[1] + jnp.where(cur == 1, _LOG_PENALTY, 0.0)
        l2 = lp_ref[2] + jnp.where(cur == 2, _LOG_PENALTY, 0.0)
        m = jnp.maximum(l0, jnp.maximum(l1, l2))
        lp_ref[0] = l0 - m
        lp_ref[1] = l1 - m
        lp_ref[2] = l2 - m

        w = lambda ref: ref[pl.ds(cur, 1)].reshape(_IN, _G3)
        b = lambda ref: ref[pl.ds(cur, 1)].reshape(1, _G3)

        def gates(gi, gh, h_prev):
            r = jax.nn.sigmoid(gi[:, :_H] + gh[:, :_H])
            z = jax.nn.sigmoid(gi[:, _H:2 * _H] + gh[:, _H:2 * _H])
            n = jnp.tanh(gi[:, 2 * _H:] + r * gh[:, 2 * _H:])
            return (1.0 - z) * n + z * h_prev

        gi0 = jnp.dot(x_t, w(wih0_ref), preferred_element_type=f32) + b(bih0_ref)
        gh0 = jnp.dot(h0b, w(whh0_ref), preferred_element_type=f32) + b(bhh0_ref)
        gh1 = jnp.dot(h1b, w(whh1_ref), preferred_element_type=f32) + b(bhh1_ref)
        h0n = gates(gi0, gh0, h0)
        gi1 = jnp.dot(h0n.astype(bf16), w(wih1_ref),
                      preferred_element_type=f32) + b(bih1_ref)
        h1n = gates(gi1, gh1, h1)

        out_ref[pl.ds(tl, 1)] = h1n.reshape(1, _B, _H)
        return (h0n, h1n)

    h0f, h1f = jax.lax.fori_loop(0, _TC, step, (h_ref[:_B], h_ref[_B:]))
    h_ref[:_B] = h0f
    h_ref[_B:] = h1f


def kernel(x, Wih_first, Wih_rest, Whh, bih, bhh, Wlw, blw, Wsel, bsel):
    f32, bf16 = jnp.float32, jnp.bfloat16
    xb = jnp.swapaxes(x, 0, 1).astype(bf16)
    wlw_t = Wlw.T.astype(bf16)
    w3sh = jnp.zeros((8, _H), f32).at[:_S].set(Wsel[:, :_H]).astype(bf16)
    w3x = jnp.zeros((8, _IN), f32).at[:_S].set(Wsel[:, _H:]).astype(bf16)
    bsel64 = jnp.zeros((8, 128), f32).at[:_S, 0].set(float(_B) * bsel)
    blw_r = blw.reshape(1, _H)
    wih0 = Wih_first.transpose(0, 2, 1).astype(bf16)
    wih1 = Wih_rest[:, 0].transpose(0, 2, 1).astype(bf16)
    whh0 = Whh[:, 0].transpose(0, 2, 1).astype(bf16)
    whh1 = Whh[:, 1].transpose(0, 2, 1).astype(bf16)
    bih0, bih1 = bih[:, 0][:, None, :], bih[:, 1][:, None, :]
    bhh0, bhh1 = bhh[:, 0][:, None, :], bhh[:, 1][:, None, :]

    full = lambda a: pl.BlockSpec(a.shape, lambda i: (0,) * a.ndim)
    outputs = pl.pallas_call(
        _body,
        grid=(_T // _TC,),
        in_specs=[pl.BlockSpec((_TC, _B, _IN), lambda i: (i, 0, 0))]
        + [full(a) for a in (wlw_t, w3sh, w3x, bsel64, blw_r,
                             wih0, whh0, bih0, bhh0, wih1, whh1, bih1, bhh1)],
        out_specs=pl.BlockSpec((_TC, _B, _H), lambda i: (i, 0, 0)),
        out_shape=jax.ShapeDtypeStruct((_T, _B, _H), f32),
        scratch_shapes=[pltpu.VMEM((2 * _B, _H), f32),
                        pltpu.SMEM((8,), f32)],
        compiler_params=pltpu.CompilerParams(
            dimension_semantics=("arbitrary",),
            vmem_limit_bytes=64 * 1024 * 1024,
        ),
    )(xb, wlw_t, w3sh, w3x, bsel64, blw_r,
      wih0, whh0, bih0, bhh0, wih1, whh1, bih1, bhh1)
    return outputs, outputs[-1]


# speedup vs baseline: 1.5439x; 1.0052x over previous
import jax
import jax.numpy as jnp
import numpy as np
from jax.experimental import pallas as pl
from jax.experimental.pallas import tpu as pltpu

_IN, _H, _L, _S = 512, 512, 2, 3
_B, _T = 64, 512
_PENALTY = 0.7
_LOG_PENALTY = float(np.log(np.float32(_PENALTY)))
_TC = 64
_UNROLL = 2
_G3 = 3 * _H


def _body(xb_ref, wlw_ref, w3sh_ref, w3x_ref, bsel64_ref, blw_ref,
          wih0_ref, whh0_ref, bih0_ref, bhh0_ref,
          wih1_ref, whh1_ref, bih1_ref, bhh1_ref,
          out_ref, h_ref, lp_ref):
    t_base = pl.program_id(0) * _TC
    f32, bf16 = jnp.float32, jnp.bfloat16

    @pl.when(t_base == 0)
    def _init():
        h_ref[...] = jnp.zeros((2 * _B, _H), f32)
        lp_ref[0] = 0.0
        lp_ref[1] = 0.0
        lp_ref[2] = 0.0

    def step(t, tl, hc):
        h0, h1 = hc
        x_t = xb_ref[pl.ds(tl, 1)].reshape(_B, _IN)

        xsum = jnp.sum(x_t.astype(f32), axis=0, keepdims=True)
        sx = jnp.sum(xsum * w3x_ref[...].astype(f32), axis=-1,
                     keepdims=True)

        h0b = h0.astype(bf16)
        h1b = h1.astype(bf16)
        hb2 = jnp.concatenate([h0b, h1b], axis=0)

        energy = jnp.dot(hb2, wlw_ref[...], preferred_element_type=f32)
        le = jnp.sum(energy + blw_ref[...], axis=-1, keepdims=True)
        le_f = le.astype(bf16).astype(f32)
        prod = le_f * hb2.astype(f32)
        sh = (prod[:_B] + prod[_B:]).astype(bf16)
        shsum = jnp.sum(sh.astype(f32), axis=0, keepdims=True)
        ssh = jnp.sum(shsum * w3sh_ref[...].astype(f32), axis=-1,
                      keepdims=True)
        srow = ssh + sx + bsel64_ref[...][:, :1]

        s0 = srow[0, 0] + lp_ref[0]
        s1 = srow[1, 0] + lp_ref[1]
        s2 = srow[2, 0] + lp_ref[2]
        cur = jnp.where((s0 >= s1) & (s0 >= s2), 0,
                        jnp.where(s1 >= s2, 1, 2)).astype(jnp.int32)
        cur = jnp.where(t == 0, 0, cur)

        l0 = lp_ref[0] + jnp.where(cur == 0, _LOG_PENALTY, 0.0)
        l1 = lp_ref[1] + jnp.where(cur == 1, _LOG_PENALTY, 0.0)
        l2 = lp_ref[2] + jnp.where(cur == 2, _LOG_PENALTY, 0.0)
        m = jnp.maximum(l0, jnp.maximum(l1, l2))
        lp_ref[0] = l0 - m
        lp_ref[1] = l1 - m
        lp_ref[2] = l2 - m

        w = lambda ref: ref[pl.ds(cur, 1)].reshape(_IN, _G3)
        b = lambda ref: ref[pl.ds(cur, 1)].reshape(1, _G3)

        def gates(gi, gh, h_prev):
            r = jax.nn.sigmoid(gi[:, :_H] + gh[:, :_H])
            z = jax.nn.sigmoid(gi[:, _H:2 * _H] + gh[:, _H:2 * _H])
            n = jnp.tanh(gi[:, 2 * _H:] + r * gh[:, 2 * _H:])
            return (1.0 - z) * n + z * h_prev

        gi0 = jnp.dot(x_t, w(wih0_ref), preferred_element_type=f32) + b(bih0_ref)
        gh0 = jnp.dot(h0b, w(whh0_ref), preferred_element_type=f32) + b(bhh0_ref)
        gh1 = jnp.dot(h1b, w(whh1_ref), preferred_element_type=f32) + b(bhh1_ref)
        h0n = gates(gi0, gh0, h0)
        gi1 = jnp.dot(h0n.astype(bf16), w(wih1_ref),
                      preferred_element_type=f32) + b(bih1_ref)
        h1n = gates(gi1, gh1, h1)

        out_ref[pl.ds(tl, 1)] = h1n.reshape(1, _B, _H)
        return (h0n, h1n)

    def pair(u, hc):
        tl = u * _UNROLL
        for k in range(_UNROLL):
            hc = step(t_base + tl + k, tl + k, hc)
        return hc

    h0f, h1f = jax.lax.fori_loop(0, _TC // _UNROLL, pair,
                                 (h_ref[:_B], h_ref[_B:]))
    h_ref[:_B] = h0f
    h_ref[_B:] = h1f


def kernel(x, Wih_first, Wih_rest, Whh, bih, bhh, Wlw, blw, Wsel, bsel):
    f32, bf16 = jnp.float32, jnp.bfloat16
    xb = jnp.swapaxes(x, 0, 1).astype(bf16)
    wlw_t = Wlw.T.astype(bf16)
    w3sh = jnp.zeros((8, _H), f32).at[:_S].set(Wsel[:, :_H]).astype(bf16)
    w3x = jnp.zeros((8, _IN), f32).at[:_S].set(Wsel[:, _H:]).astype(bf16)
    bsel64 = jnp.zeros((8, 128), f32).at[:_S, 0].set(float(_B) * bsel)
    blw_r = blw.reshape(1, _H)
    wih0 = Wih_first.transpose(0, 2, 1).astype(bf16)
    wih1 = Wih_rest[:, 0].transpose(0, 2, 1).astype(bf16)
    whh0 = Whh[:, 0].transpose(0, 2, 1).astype(bf16)
    whh1 = Whh[:, 1].transpose(0, 2, 1).astype(bf16)
    bih0, bih1 = bih[:, 0][:, None, :], bih[:, 1][:, None, :]
    bhh0, bhh1 = bhh[:, 0][:, None, :], bhh[:, 1][:, None, :]

    full = lambda a: pl.BlockSpec(a.shape, lambda i: (0,) * a.ndim)
    outputs = pl.pallas_call(
        _body,
        grid=(_T // _TC,),
        in_specs=[pl.BlockSpec((_TC, _B, _IN), lambda i: (i, 0, 0))]
        + [full(a) for a in (wlw_t, w3sh, w3x, bsel64, blw_r,
                             wih0, whh0, bih0, bhh0, wih1, whh1, bih1, bhh1)],
        out_specs=pl.BlockSpec((_TC, _B, _H), lambda i: (i, 0, 0)),
        out_shape=jax.ShapeDtypeStruct((_T, _B, _H), f32),
        scratch_shapes=[pltpu.VMEM((2 * _B, _H), f32),
                        pltpu.SMEM((8,), f32)],
        compiler_params=pltpu.CompilerParams(
            dimension_semantics=("arbitrary",),
            vmem_limit_bytes=64 * 1024 * 1024,
        ),
    )(xb, wlw_t, w3sh, w3x, bsel64, blw_r,
      wih0, whh0, bih0, bhh0, wih1, whh1, bih1, bhh1)
    return outputs, outputs[-1]
